# baseline (speedup 1.0000x reference)
import functools

import jax
import jax.numpy as jnp
from jax import lax
from jax.experimental import pallas as pl
from jax.experimental.pallas import tpu as pltpu
from jax.experimental.pallas import tpu_sc as plsc

N_POINTS = 100000
N_L1 = 20000
N_L2 = 4000
E_L2 = 64000
HID = 64
N_CLASSES = 20

NPp = 102400
NL1p = 20480
NL2p = 4096
NW = 32
CH = 128

_mesh = plsc.VectorSubcoreMesh(core_axis_name="c", subcore_axis_name="s")
_sc_params = pltpu.CompilerParams(
    needs_layout_passes=False, use_tc_tiling_on_sc=False
)


def _wid():
    return lax.axis_index("s") * 2 + lax.axis_index("c")


def _ci(n, cap=8):
    nch = n // CH
    k = 1
    for cand in range(2, cap + 1):
        if nch % cand == 0:
            k = cand
    return k * CH


def _zero_vmem(ref, rows):
    z16 = jnp.zeros((16,), jnp.float32)

    @pl.loop(0, rows)
    def _(r):
        for q in range(HID // 16):
            ref[r, pl.ds(q * 16, 16)] = z16


def _sc_rels(xin_flat, c1t_flat, c2t_flat, lbl1, lbl2, src, dst):
    ci1 = _ci(NPp)
    ci2 = _ci(NL1p)
    ci3 = _ci(E_L2)

    @functools.partial(
        pl.kernel,
        out_type=(
            jax.ShapeDtypeStruct((NPp * 4,), jnp.float32),
            jax.ShapeDtypeStruct((NL1p * 4,), jnp.float32),
            jax.ShapeDtypeStruct((E_L2 * 4,), jnp.float32),
        ),
        mesh=_mesh,
        compiler_params=_sc_params,
        scratch_types=[
            pltpu.VMEM((NL1p * 4,), jnp.float32),
            pltpu.VMEM((NL2p * 4,), jnp.float32),
            pltpu.VMEM((max(ci1, ci2, ci3),), jnp.int32),
            pltpu.VMEM((ci3,), jnp.int32),
            pltpu.VMEM((max(ci1, ci2) * 4,), jnp.float32),
            pltpu.VMEM((max(ci1, ci2, ci3) * 4,), jnp.float32),
        ],
    )
    def k(x_hbm, t1_hbm, t2_hbm, l1_hbm, l2_hbm, s_hbm, d_hbm,
          o1_hbm, o2_hbm, o3_hbm, tbl1, tbl2, idxv, dv, xv, ov):
        wid = _wid()
        pltpu.sync_copy(t1_hbm, tbl1)
        pltpu.sync_copy(t2_hbm, tbl2)
        iota = lax.iota(jnp.int32, 16)
        idiv = lax.shift_right_logical(iota, 2)
        imod = lax.bitwise_and(iota, 3)

        def rel_phase(ci, n, a_hbm, idx_hbm, out_hbm, tblv):
            @pl.loop(wid, n // ci, step=NW)
            def _(c):
                base = c * ci
                pltpu.sync_copy(idx_hbm.at[pl.ds(base, ci)], idxv.at[pl.ds(0, ci)])
                pltpu.sync_copy(a_hbm.at[pl.ds(base * 4, ci * 4)],
                                xv.at[pl.ds(0, ci * 4)])
                for v in range(ci * 4 // 16):
                    rowsel = plsc.load_gather(idxv, [jnp.int32(v * 4) + idiv])
                    tidx = rowsel * 4 + imod
                    tv = plsc.load_gather(tblv, [tidx])
                    ov[pl.ds(v * 16, 16)] = xv[pl.ds(v * 16, 16)] - tv
                pltpu.sync_copy(ov.at[pl.ds(0, ci * 4)],
                                out_hbm.at[pl.ds(base * 4, ci * 4)])

        rel_phase(ci1, NPp, x_hbm, l1_hbm, o1_hbm, tbl1)
        rel_phase(ci2, NL1p, t1_hbm, l2_hbm, o2_hbm, tbl2)

        @pl.loop(wid, E_L2 // ci3, step=NW)
        def _(c):
            base = c * ci3
            pltpu.sync_copy(s_hbm.at[pl.ds(base, ci3)], idxv.at[pl.ds(0, ci3)])
            pltpu.sync_copy(d_hbm.at[pl.ds(base, ci3)], dv)
            for v in range(ci3 * 4 // 16):
                lane = jnp.int32(v * 4) + idiv
                si = plsc.load_gather(idxv, [lane]) * 4 + imod
                di = plsc.load_gather(dv, [lane]) * 4 + imod
                ov[pl.ds(v * 16, 16)] = (
                    plsc.load_gather(tbl2, [si]) - plsc.load_gather(tbl2, [di])
                )
            pltpu.sync_copy(ov.at[pl.ds(0, ci3 * 4)],
                            o3_hbm.at[pl.ds(base * 4, ci3 * 4)])

    return k(xin_flat, c1t_flat, c2t_flat, lbl1, lbl2, src, dst)


def _sc_scatter_add(vals, idx2):
    n = vals.shape[0]
    ci = _ci(n, cap=2)
    kk = ci // CH
    nchunks = n // ci
    rows_per_sub = NL1p // 16

    @functools.partial(
        pl.kernel,
        out_type=jax.ShapeDtypeStruct((2, NL1p, HID), jnp.float32),
        mesh=_mesh,
        compiler_params=_sc_params,
        scratch_types=[
            pltpu.VMEM_SHARED((NL1p, HID), jnp.float32),
            pltpu.VMEM((ci, HID), jnp.float32),
            pltpu.VMEM((kk, CH), jnp.int32),
        ],
    )
    def k(vals_hbm, idx_hbm, out_hbm, acc, hv, idxv):
        cid = lax.axis_index("c")
        sid = lax.axis_index("s")
        wid = sid * 2 + cid
        _zero_vmem(hv, CH)
        for i in range(rows_per_sub // CH):
            pltpu.sync_copy(
                hv.at[pl.ds(0, CH), :],
                acc.at[pl.ds(sid * rows_per_sub + i * CH, CH), :],
            )
        plsc.subcore_barrier()

        @pl.loop(wid, nchunks, step=NW)
        def _(c):
            pltpu.sync_copy(idx_hbm.at[pl.ds(c * kk, kk), :], idxv)
            pltpu.sync_copy(vals_hbm.at[pl.ds(c * ci, ci), :], hv)
            for i in range(kk):
                pltpu.sync_copy(
                    hv.at[pl.ds(i * CH, CH), :], acc.at[idxv.at[i]], add=True
                )

        plsc.subcore_barrier()
        pltpu.sync_copy(
            acc.at[pl.ds(sid * rows_per_sub, rows_per_sub), :],
            out_hbm.at[cid, pl.ds(sid * rows_per_sub, rows_per_sub), :],
        )

    return k(vals, idx2)


def _sc_gather_rows(tbl, idx2):
    n = idx2.shape[0] * CH
    ci = _ci(n)
    kk = ci // CH
    nchunks = n // ci

    @functools.partial(
        pl.kernel,
        out_type=jax.ShapeDtypeStruct((n, HID), jnp.float32),
        mesh=_mesh,
        compiler_params=_sc_params,
        scratch_types=[
            pltpu.VMEM((kk, CH), jnp.int32),
            pltpu.VMEM((ci, HID), jnp.float32),
            pltpu.SemaphoreType.DMA,
        ],
    )
    def k(tbl_hbm, idx_hbm, out_hbm, idxv, rows, sem):
        wid = _wid()

        @pl.loop(wid, nchunks, step=NW)
        def _(c):
            pltpu.sync_copy(idx_hbm.at[pl.ds(c * kk, kk), :], idxv)
            descs = [
                pltpu.async_copy(
                    tbl_hbm.at[idxv.at[i]], rows.at[pl.ds(i * CH, CH), :], sem
                )
                for i in range(kk)
            ]
            for d in descs:
                d.wait()
            pltpu.sync_copy(rows, out_hbm.at[pl.ds(c * ci, ci), :])

    return k(tbl, idx2)


def _sc_segmax(vals, idx):
    n = vals.shape[0]
    ci = _ci(n, cap=8)
    nchunks = n // ci
    CAP = 2048

    @functools.partial(
        pl.kernel,
        out_type=jax.ShapeDtypeStruct((NL2p, HID), jnp.float32),
        mesh=_mesh,
        compiler_params=_sc_params,
        scratch_types=[
            pltpu.VMEM((CH + 16, HID), jnp.float32),
            pltpu.VMEM((CH, HID), jnp.float32),
            pltpu.VMEM((ci,), jnp.int32),
            pltpu.VMEM(((CAP + CH) // CH, CH), jnp.int32),
            pltpu.VMEM((CAP + 16,), jnp.int32),
            pltpu.SemaphoreType.DMA,
        ],
    )
    def k(vals_hbm, idx_hbm, out_hbm, acc, rbuf, idxv, mrow, mdst, sem):
        wid = _wid()
        lo = wid * CH
        _zero_vmem(acc, CH + 16)
        zi = jnp.zeros((16,), jnp.int32)
        for r in range((CAP + CH) // CH):
            for g in range(CH // 16):
                mrow[r, pl.ds(g * 16, 16)] = zi
        iota = lax.iota(jnp.int32, 16)
        trash = jnp.full((16,), CH, jnp.int32)

        def flush(cnt):
            mdst[pl.ds(cnt, 16)] = trash
            ng = lax.div(cnt + 16, 16)

            @pl.loop(0, lax.div(ng + 7, 8))
            def _(s):
                pltpu.async_copy(vals_hbm.at[mrow.at[s]], rbuf, sem).wait()
                gs = jnp.minimum(8, ng - s * 8)

                @pl.loop(0, gs)
                def _(g):
                    dv = mdst[pl.ds(s * CH + g * 16, 16)]
                    for j in range(16):
                        d = dv[j]
                        for q in range(HID // 16):
                            sl = pl.ds(q * 16, 16)
                            acc[d, sl] = jnp.maximum(
                                acc[d, sl], rbuf[g * 16 + j, sl]
                            )

        @pl.loop(0, nchunks, init_carry=jnp.int32(0))
        def scan(c, cnt_in):
            base = c * ci
            pltpu.sync_copy(idx_hbm.at[pl.ds(base, ci)], idxv)
            cntv = jnp.full((16,), cnt_in, jnp.int32)
            for v in range(ci // 16):
                lbl = idxv[pl.ds(v * 16, 16)]
                rel = lbl - lo
                cmask = (rel >= 0) & (rel < CH)
                pos = cntv + plsc.cumsum(cmask.astype(jnp.int32)) - 1
                rowid = base + v * 16 + iota
                plsc.store_scatter(
                    mrow,
                    [lax.shift_right_logical(pos, 7), lax.bitwise_and(pos, CH - 1)],
                    rowid,
                    mask=cmask,
                )
                plsc.store_scatter(mdst, [pos], rel, mask=cmask)
                cntv = cntv + plsc.all_reduce_population_count(cmask)
            cnt = cntv[0]
            do_flush = cnt > CAP - ci

            @pl.when(do_flush)
            def _():
                flush(cnt)

            return jnp.where(do_flush, jnp.int32(0), cnt)

        flush(scan)
        pltpu.sync_copy(acc.at[pl.ds(0, CH), :], out_hbm.at[pl.ds(lo, CH), :])

    return k(vals, idx)


def _full(shape):
    return pl.BlockSpec(shape, lambda i: tuple(0 for _ in shape))


def _rows(bm, ncol):
    return pl.BlockSpec((bm, ncol), lambda i: (i, 0))


def _tc_encode(x, w, b):
    n = x.shape[0]
    bm = 1024

    def body(x_ref, w_ref, b_ref, o_ref):
        o_ref[...] = jnp.maximum(
            jnp.dot(x_ref[...], w_ref[...], preferred_element_type=jnp.float32)
            + b_ref[...],
            0.0,
        )

    return pl.pallas_call(
        body,
        grid=(n // bm,),
        in_specs=[_rows(bm, 4), _full((4, HID)), _full((1, HID))],
        out_specs=_rows(bm, HID),
        out_shape=jax.ShapeDtypeStruct((n, HID), jnp.float32),
    )(x, w, b)


def _tc_l1l3(aggp, rel2, w1, b1, w2, b2, w3a, w3b, b3):
    n = aggp.shape[1]
    bm = 512

    def body(a_ref, r_ref, w1_ref, b1_ref, w2_ref, b2_ref,
             w3a_ref, w3b_ref, b3_ref, o_ref):
        x = a_ref[0] + a_ref[1]
        h = jnp.maximum(
            jnp.dot(x, w1_ref[...], preferred_element_type=jnp.float32) + b1_ref[...],
            0.0,
        )
        f1 = jnp.maximum(
            jnp.dot(h, w2_ref[...], preferred_element_type=jnp.float32) + b2_ref[...],
            0.0,
        )
        o_ref[...] = jnp.maximum(
            jnp.dot(f1, w3a_ref[...], preferred_element_type=jnp.float32)
            + jnp.dot(r_ref[...], w3b_ref[...], preferred_element_type=jnp.float32)
            + b3_ref[...],
            0.0,
        )

    return pl.pallas_call(
        body,
        grid=(n // bm,),
        in_specs=[
            pl.BlockSpec((2, bm, HID), lambda i: (0, i, 0)),
            _rows(bm, 4),
            _full((HID, HID)),
            _full((1, HID)),
            _full((HID, HID)),
            _full((1, HID)),
            _full((HID, HID)),
            _full((4, HID)),
            _full((1, HID)),
        ],
        out_specs=_rows(bm, HID),
        out_shape=jax.ShapeDtypeStruct((n, HID), jnp.float32),
    )(aggp, rel2, w1, b1, w2, b2, w3a, w3b, b3)


def _tc_mix(g, r, wa, wb, b, relu=True, bm=512):
    n = g.shape[0]

    def body(g_ref, r_ref, wa_ref, wb_ref, b_ref, o_ref):
        x = (
            jnp.dot(g_ref[...], wa_ref[...], preferred_element_type=jnp.float32)
            + jnp.dot(r_ref[...], wb_ref[...], preferred_element_type=jnp.float32)
            + b_ref[...]
        )
        o_ref[...] = jnp.maximum(x, 0.0) if relu else x

    return pl.pallas_call(
        body,
        grid=(n // bm,),
        in_specs=[
            _rows(bm, HID),
            _rows(bm, 4),
            _full((HID, HID)),
            _full((4, HID)),
            _full((1, HID)),
        ],
        out_specs=_rows(bm, HID),
        out_shape=jax.ShapeDtypeStruct((n, HID), jnp.float32),
    )(g, r, wa, wb, b)


def _tc_matmul_bias(x, w, b):
    n = x.shape[0]

    def body(x_ref, w_ref, b_ref, o_ref):
        o_ref[...] = (
            jnp.dot(x_ref[...], w_ref[...], preferred_element_type=jnp.float32)
            + b_ref[...]
        )

    return pl.pallas_call(
        body,
        grid=(1,),
        in_specs=[_rows(n, HID), _full((HID, HID)), _full((1, HID))],
        out_specs=_rows(n, HID),
        out_shape=jax.ShapeDtypeStruct((n, HID), jnp.float32),
    )(x, w, b)


def _tc_update(x, m, wg, bg, wfx_next, bf_next):
    n = x.shape[0]

    def body(x_ref, m_ref, wg_ref, bg_ref, wn_ref, bn_ref, o1_ref, o2_ref):
        x1 = x_ref[...] + jnp.maximum(
            jnp.dot(m_ref[...], wg_ref[...], preferred_element_type=jnp.float32)
            + bg_ref[...],
            0.0,
        )
        o1_ref[...] = x1
        o2_ref[...] = (
            jnp.dot(x1, wn_ref[...], preferred_element_type=jnp.float32) + bn_ref[...]
        )

    return pl.pallas_call(
        body,
        grid=(1,),
        in_specs=[
            _rows(n, HID),
            _rows(n, HID),
            _full((HID, HID)),
            _full((1, HID)),
            _full((HID, HID)),
            _full((1, HID)),
        ],
        out_specs=[_rows(n, HID), _rows(n, HID)],
        out_shape=[
            jax.ShapeDtypeStruct((n, HID), jnp.float32),
            jax.ShapeDtypeStruct((n, HID), jnp.float32),
        ],
    )(x, m, wg, bg, wfx_next, bf_next)


def _tc_update_last(x, m, wg, bg):
    n = x.shape[0]

    def body(x_ref, m_ref, wg_ref, bg_ref, o_ref):
        o_ref[...] = x_ref[...] + jnp.maximum(
            jnp.dot(m_ref[...], wg_ref[...], preferred_element_type=jnp.float32)
            + bg_ref[...],
            0.0,
        )

    return pl.pallas_call(
        body,
        grid=(1,),
        in_specs=[_rows(n, HID), _rows(n, HID), _full((HID, HID)), _full((1, HID))],
        out_specs=_rows(n, HID),
        out_shape=jax.ShapeDtypeStruct((n, HID), jnp.float32),
    )(x, m, wg, bg)


def _tc_addmm(g, r, wb):
    n = g.shape[0]
    bm = 512

    def body(g_ref, r_ref, wb_ref, o_ref):
        o_ref[...] = g_ref[...] + jnp.dot(
            r_ref[...], wb_ref[...], preferred_element_type=jnp.float32
        )

    return pl.pallas_call(
        body,
        grid=(n // bm,),
        in_specs=[_rows(bm, HID), _rows(bm, 4), _full((4, HID))],
        out_specs=_rows(bm, HID),
        out_shape=jax.ShapeDtypeStruct((n, HID), jnp.float32),
    )(g, r, wb)


def _tc_final(g, r, wa, wb, b, wc, bc):
    n = g.shape[0]
    bm = 1024

    def body(g_ref, r_ref, wa_ref, wb_ref, b_ref, wc_ref, bc_ref, o_ref):
        f = jnp.maximum(
            jnp.dot(g_ref[...], wa_ref[...], preferred_element_type=jnp.float32)
            + jnp.dot(r_ref[...], wb_ref[...], preferred_element_type=jnp.float32)
            + b_ref[...],
            0.0,
        )
        o_ref[...] = (
            jnp.dot(f, wc_ref[...], preferred_element_type=jnp.float32) + bc_ref[...]
        )

    return pl.pallas_call(
        body,
        grid=(n // bm,),
        in_specs=[
            _rows(bm, HID),
            _rows(bm, 4),
            _full((HID, HID)),
            _full((4, HID)),
            _full((1, HID)),
            _full((HID, N_CLASSES)),
            _full((1, N_CLASSES)),
        ],
        out_specs=_rows(bm, N_CLASSES),
        out_shape=jax.ShapeDtypeStruct((n, N_CLASSES), jnp.float32),
    )(g, r, wa, wb, b, wc, bc)


def kernel(remission, points, l1_cluster_centers, l2_cluster_centers,
           l1_edges, l2_edges, l1_labels, l2_labels,
           W1a, b1a, W1b, b1b, W1c, b1c, W3, b3,
           Wf4, bf4, Wg4, bg4, Wf41, bf41, Wg41, bg41,
           W5, b5, W7, b7, Wc, bc):
    f32 = jnp.float32
    xin = jnp.zeros((NPp, 4), f32).at[:N_POINTS, 0:1].set(remission)
    xin = xin.at[:N_POINTS, 1:4].set(points)
    c1t = jnp.zeros((NL1p, 4), f32).at[:N_L1, 1:4].set(l1_cluster_centers)
    c2t = jnp.zeros((NL2p, 4), f32).at[:N_L2, 1:4].set(l2_cluster_centers)
    c1_4 = jnp.zeros((NL1p, 4), f32).at[:N_L1, 1:4].set(l1_cluster_centers)
    lbl1 = jnp.full((NPp,), N_L1, jnp.int32).at[:N_POINTS].set(l1_labels)
    lbl2_g = jnp.zeros((NL1p,), jnp.int32).at[:N_L1].set(l2_labels)
    lbl2_m = jnp.full((NL1p,), -1, jnp.int32).at[:N_L1].set(l2_labels)
    src = l2_edges[0]
    dst = l2_edges[1]
    lbl1_2d = lbl1.reshape(NPp // CH, CH)
    lbl2g_2d = lbl2_g.reshape(NL1p // CH, CH)
    src_2d = src.reshape(E_L2 // CH, CH)

    def split67(w):
        return w[:HID], jnp.concatenate([jnp.zeros((1, HID), f32), w[HID:]], axis=0)

    W3a, W3b = split67(W3)
    W5a, W5b = split67(W5)
    W7a, W7b = split67(W7)
    Wf4x, Wf4e = split67(Wf4)
    Wf41x, Wf41e = split67(Wf41)
    b1a_, b1b_, b1c_, b3_ = (b.reshape(1, HID) for b in (b1a, b1b, b1c, b3))
    bf4_, bg4_, bf41_, bg41_ = (b.reshape(1, HID) for b in (bf4, bg4, bf41, bg41))
    b5_, b7_ = b5.reshape(1, HID), b7.reshape(1, HID)
    bc_ = bc.reshape(1, N_CLASSES)

    rel_in, rel2, rel_e = _sc_rels(
        xin.reshape(-1), c1t.reshape(-1), c2t.reshape(-1),
        lbl1, lbl2_g, src, dst,
    )
    rel_in = rel_in.reshape(NPp, 4)
    rel2 = rel2.reshape(NL1p, 4)
    rel_e = rel_e.reshape(E_L2, 4)

    h = _tc_encode(rel_in, W1a, b1a_)
    aggp = _sc_scatter_add(h, lbl1_2d)
    m = _tc_l1l3(aggp, rel2, W1b, b1b_, W1c, b1c_, W3a, W3b, b3_)
    x = _sc_segmax(m, lbl2_m)


    p4 = _tc_matmul_bias(x, Wf4x, bf4_)
    z4 = _tc_addmm(_sc_gather_rows(p4, src_2d), rel_e, Wf4e)
    m4 = _sc_segmax(z4, dst)
    x, p41 = _tc_update(x, m4, Wg4, bg4_, Wf41x, bf41_)
    z41 = _tc_addmm(_sc_gather_rows(p41, src_2d), rel_e, Wf41e)
    m41 = _sc_segmax(z41, dst)
    x = _tc_update_last(x, m41, Wg41, bg41_)

    g4 = _sc_gather_rows(x, lbl2g_2d)
    f5 = _tc_mix(g4, rel2, W5a, W5b, b5_)
    g5 = _sc_gather_rows(f5, lbl1_2d)
    out = _tc_final(g5, rel_in, W7a, W7b, b7_, Wc, bc_)
    return out[:N_POINTS]


# speedup vs baseline: 1.0069x; 1.0069x over previous
import functools

import jax
import jax.numpy as jnp
from jax import lax
from jax.experimental import pallas as pl
from jax.experimental.pallas import tpu as pltpu
from jax.experimental.pallas import tpu_sc as plsc

N_POINTS = 100000
N_L1 = 20000
N_L2 = 4000
E_L2 = 64000
HID = 64
N_CLASSES = 20

NPp = 102400
NL1p = 20480
NL2p = 4096
NW = 32
CH = 128

_mesh = plsc.VectorSubcoreMesh(core_axis_name="c", subcore_axis_name="s")
_sc_params = pltpu.CompilerParams(
    needs_layout_passes=False, use_tc_tiling_on_sc=False
)


def _wid():
    return lax.axis_index("s") * 2 + lax.axis_index("c")


def _ci(n, cap=8):
    nch = n // CH
    k = 1
    for cand in range(2, cap + 1):
        if nch % cand == 0:
            k = cand
    return k * CH


def _zero_vmem(ref, rows):
    z16 = jnp.zeros((16,), jnp.float32)

    @pl.loop(0, rows)
    def _(r):
        for q in range(HID // 16):
            ref[r, pl.ds(q * 16, 16)] = z16


def _sc_rel_label(n, t, a_flat, tbl_flat, idx):
    ci = _ci(n)
    nchunks = n // ci

    @functools.partial(
        pl.kernel,
        out_type=jax.ShapeDtypeStruct((n * 4,), jnp.float32),
        mesh=_mesh,
        compiler_params=_sc_params,
        scratch_types=[
            pltpu.VMEM((t * 4,), jnp.float32),
            pltpu.VMEM((ci,), jnp.int32),
            pltpu.VMEM((ci * 4,), jnp.float32),
            pltpu.VMEM((ci * 4,), jnp.float32),
        ],
    )
    def k(a_hbm, tbl_hbm, idx_hbm, out_hbm, tblv, idxv, xv, ov):
        wid = _wid()
        pltpu.sync_copy(tbl_hbm, tblv)
        iota = lax.iota(jnp.int32, 16)
        idiv = lax.shift_right_logical(iota, 2)
        imod = lax.bitwise_and(iota, 3)

        @pl.loop(wid, nchunks, step=NW)
        def _(c):
            base = c * ci
            pltpu.sync_copy(idx_hbm.at[pl.ds(base, ci)], idxv)
            pltpu.sync_copy(a_hbm.at[pl.ds(base * 4, ci * 4)], xv)
            for v in range(ci * 4 // 16):
                rowsel = plsc.load_gather(idxv, [jnp.int32(v * 4) + idiv])
                tidx = rowsel * 4 + imod
                tv = plsc.load_gather(tblv, [tidx])
                ov[pl.ds(v * 16, 16)] = xv[pl.ds(v * 16, 16)] - tv
            pltpu.sync_copy(ov, out_hbm.at[pl.ds(base * 4, ci * 4)])

    return k(a_flat, tbl_flat, idx)


def _sc_rel_edges(n, t, tbl_flat, src, dst):
    ci = _ci(n)
    nchunks = n // ci

    @functools.partial(
        pl.kernel,
        out_type=jax.ShapeDtypeStruct((n * 4,), jnp.float32),
        mesh=_mesh,
        compiler_params=_sc_params,
        scratch_types=[
            pltpu.VMEM((t * 4,), jnp.float32),
            pltpu.VMEM((ci,), jnp.int32),
            pltpu.VMEM((ci,), jnp.int32),
            pltpu.VMEM((ci * 4,), jnp.float32),
        ],
    )
    def k(tbl_hbm, src_hbm, dst_hbm, out_hbm, tblv, sv, dv, ov):
        wid = _wid()
        pltpu.sync_copy(tbl_hbm, tblv)
        iota = lax.iota(jnp.int32, 16)
        idiv = lax.shift_right_logical(iota, 2)
        imod = lax.bitwise_and(iota, 3)

        @pl.loop(wid, nchunks, step=NW)
        def _(c):
            base = c * ci
            pltpu.sync_copy(src_hbm.at[pl.ds(base, ci)], sv)
            pltpu.sync_copy(dst_hbm.at[pl.ds(base, ci)], dv)
            for v in range(ci * 4 // 16):
                lane = jnp.int32(v * 4) + idiv
                si = plsc.load_gather(sv, [lane]) * 4 + imod
                di = plsc.load_gather(dv, [lane]) * 4 + imod
                ov[pl.ds(v * 16, 16)] = (
                    plsc.load_gather(tblv, [si]) - plsc.load_gather(tblv, [di])
                )
            pltpu.sync_copy(ov, out_hbm.at[pl.ds(base * 4, ci * 4)])

    return k(tbl_flat, src, dst)


def _sc_scatter_add(vals, idx2):
    n = vals.shape[0]
    ci = _ci(n, cap=2)
    kk = ci // CH
    nchunks = n // ci
    rows_per_sub = NL1p // 16

    @functools.partial(
        pl.kernel,
        out_type=jax.ShapeDtypeStruct((2, NL1p, HID), jnp.float32),
        mesh=_mesh,
        compiler_params=_sc_params,
        scratch_types=[
            pltpu.VMEM_SHARED((NL1p, HID), jnp.float32),
            pltpu.VMEM((ci, HID), jnp.float32),
            pltpu.VMEM((kk, CH), jnp.int32),
        ],
    )
    def k(vals_hbm, idx_hbm, out_hbm, acc, hv, idxv):
        cid = lax.axis_index("c")
        sid = lax.axis_index("s")
        wid = sid * 2 + cid
        _zero_vmem(hv, CH)
        for i in range(rows_per_sub // CH):
            pltpu.sync_copy(
                hv.at[pl.ds(0, CH), :],
                acc.at[pl.ds(sid * rows_per_sub + i * CH, CH), :],
            )
        plsc.subcore_barrier()

        @pl.loop(wid, nchunks, step=NW)
        def _(c):
            pltpu.sync_copy(idx_hbm.at[pl.ds(c * kk, kk), :], idxv)
            pltpu.sync_copy(vals_hbm.at[pl.ds(c * ci, ci), :], hv)
            for i in range(kk):
                pltpu.sync_copy(
                    hv.at[pl.ds(i * CH, CH), :], acc.at[idxv.at[i]], add=True
                )

        plsc.subcore_barrier()
        pltpu.sync_copy(
            acc.at[pl.ds(sid * rows_per_sub, rows_per_sub), :],
            out_hbm.at[cid, pl.ds(sid * rows_per_sub, rows_per_sub), :],
        )

    return k(vals, idx2)


def _sc_gather_rows(tbl, idx2):
    n = idx2.shape[0] * CH
    ci = _ci(n)
    kk = ci // CH
    nchunks = n // ci

    @functools.partial(
        pl.kernel,
        out_type=jax.ShapeDtypeStruct((n, HID), jnp.float32),
        mesh=_mesh,
        compiler_params=_sc_params,
        scratch_types=[
            pltpu.VMEM((kk, CH), jnp.int32),
            pltpu.VMEM((ci, HID), jnp.float32),
            pltpu.SemaphoreType.DMA,
        ],
    )
    def k(tbl_hbm, idx_hbm, out_hbm, idxv, rows, sem):
        wid = _wid()

        @pl.loop(wid, nchunks, step=NW)
        def _(c):
            pltpu.sync_copy(idx_hbm.at[pl.ds(c * kk, kk), :], idxv)
            descs = [
                pltpu.async_copy(
                    tbl_hbm.at[idxv.at[i]], rows.at[pl.ds(i * CH, CH), :], sem
                )
                for i in range(kk)
            ]
            for d in descs:
                d.wait()
            pltpu.sync_copy(rows, out_hbm.at[pl.ds(c * ci, ci), :])

    return k(tbl, idx2)


def _sc_segmax(vals, idx):
    n = vals.shape[0]
    ci = _ci(n, cap=8)
    nchunks = n // ci
    CAP = 2048

    @functools.partial(
        pl.kernel,
        out_type=jax.ShapeDtypeStruct((NL2p, HID), jnp.float32),
        mesh=_mesh,
        compiler_params=_sc_params,
        scratch_types=[
            pltpu.VMEM((CH + 16, HID), jnp.float32),
            pltpu.VMEM((CH, HID), jnp.float32),
            pltpu.VMEM((ci,), jnp.int32),
            pltpu.VMEM(((CAP + CH) // CH, CH), jnp.int32),
            pltpu.VMEM((CAP + 16,), jnp.int32),
            pltpu.SemaphoreType.DMA,
        ],
    )
    def k(vals_hbm, idx_hbm, out_hbm, acc, rbuf, idxv, mrow, mdst, sem):
        wid = _wid()
        lo = wid * CH
        _zero_vmem(acc, CH + 16)
        zi = jnp.zeros((16,), jnp.int32)
        for r in range((CAP + CH) // CH):
            for g in range(CH // 16):
                mrow[r, pl.ds(g * 16, 16)] = zi
        iota = lax.iota(jnp.int32, 16)
        trash = jnp.full((16,), CH, jnp.int32)

        def flush(cnt):
            mdst[pl.ds(cnt, 16)] = trash
            ng = lax.div(cnt + 16, 16)

            @pl.loop(0, lax.div(ng + 7, 8))
            def _(s):
                pltpu.async_copy(vals_hbm.at[mrow.at[s]], rbuf, sem).wait()
                gs = jnp.minimum(8, ng - s * 8)

                @pl.loop(0, gs)
                def _(g):
                    dv = mdst[pl.ds(s * CH + g * 16, 16)]
                    for j in range(16):
                        d = dv[j]
                        for q in range(HID // 16):
                            sl = pl.ds(q * 16, 16)
                            acc[d, sl] = jnp.maximum(
                                acc[d, sl], rbuf[g * 16 + j, sl]
                            )

        @pl.loop(0, nchunks, init_carry=jnp.int32(0))
        def scan(c, cnt_in):
            base = c * ci
            pltpu.sync_copy(idx_hbm.at[pl.ds(base, ci)], idxv)
            cntv = jnp.full((16,), cnt_in, jnp.int32)
            for v in range(ci // 16):
                lbl = idxv[pl.ds(v * 16, 16)]
                rel = lbl - lo
                cmask = (rel >= 0) & (rel < CH)
                pos = cntv + plsc.cumsum(cmask.astype(jnp.int32)) - 1
                rowid = base + v * 16 + iota
                plsc.store_scatter(
                    mrow,
                    [lax.shift_right_logical(pos, 7), lax.bitwise_and(pos, CH - 1)],
                    rowid,
                    mask=cmask,
                )
                plsc.store_scatter(mdst, [pos], rel, mask=cmask)
                cntv = cntv + plsc.all_reduce_population_count(cmask)
            cnt = cntv[0]
            do_flush = cnt > CAP - ci

            @pl.when(do_flush)
            def _():
                flush(cnt)

            return jnp.where(do_flush, jnp.int32(0), cnt)

        flush(scan)
        pltpu.sync_copy(acc.at[pl.ds(0, CH), :], out_hbm.at[pl.ds(lo, CH), :])

    return k(vals, idx)


def _full(shape):
    return pl.BlockSpec(shape, lambda i: tuple(0 for _ in shape))


def _rows(bm, ncol):
    return pl.BlockSpec((bm, ncol), lambda i: (i, 0))


def _tc_encode(x, w, b):
    n = x.shape[0]
    bm = 1024

    def body(x_ref, w_ref, b_ref, o_ref):
        o_ref[...] = jnp.maximum(
            jnp.dot(x_ref[...], w_ref[...], preferred_element_type=jnp.float32)
            + b_ref[...],
            0.0,
        )

    return pl.pallas_call(
        body,
        grid=(n // bm,),
        in_specs=[_rows(bm, 4), _full((4, HID)), _full((1, HID))],
        out_specs=_rows(bm, HID),
        out_shape=jax.ShapeDtypeStruct((n, HID), jnp.float32),
    )(x, w, b)


def _tc_mlp2(aggp, w1, b1, w2, b2):
    n = aggp.shape[1]
    bm = 512

    def body(a_ref, w1_ref, b1_ref, w2_ref, b2_ref, o_ref):
        x = a_ref[0] + a_ref[1]
        h = jnp.maximum(
            jnp.dot(x, w1_ref[...], preferred_element_type=jnp.float32) + b1_ref[...],
            0.0,
        )
        o_ref[...] = jnp.maximum(
            jnp.dot(h, w2_ref[...], preferred_element_type=jnp.float32) + b2_ref[...],
            0.0,
        )

    return pl.pallas_call(
        body,
        grid=(n // bm,),
        in_specs=[
            pl.BlockSpec((2, bm, HID), lambda i: (0, i, 0)),
            _full((HID, HID)),
            _full((1, HID)),
            _full((HID, HID)),
            _full((1, HID)),
        ],
        out_specs=_rows(bm, HID),
        out_shape=jax.ShapeDtypeStruct((n, HID), jnp.float32),
    )(aggp, w1, b1, w2, b2)


def _tc_mix(g, r, wa, wb, b, relu=True, bm=512):
    n = g.shape[0]

    def body(g_ref, r_ref, wa_ref, wb_ref, b_ref, o_ref):
        x = (
            jnp.dot(g_ref[...], wa_ref[...], preferred_element_type=jnp.float32)
            + jnp.dot(r_ref[...], wb_ref[...], preferred_element_type=jnp.float32)
            + b_ref[...]
        )
        o_ref[...] = jnp.maximum(x, 0.0) if relu else x

    return pl.pallas_call(
        body,
        grid=(n // bm,),
        in_specs=[
            _rows(bm, HID),
            _rows(bm, 4),
            _full((HID, HID)),
            _full((4, HID)),
            _full((1, HID)),
        ],
        out_specs=_rows(bm, HID),
        out_shape=jax.ShapeDtypeStruct((n, HID), jnp.float32),
    )(g, r, wa, wb, b)


def _tc_matmul_bias(x, w, b):
    n = x.shape[0]

    def body(x_ref, w_ref, b_ref, o_ref):
        o_ref[...] = (
            jnp.dot(x_ref[...], w_ref[...], preferred_element_type=jnp.float32)
            + b_ref[...]
        )

    return pl.pallas_call(
        body,
        grid=(1,),
        in_specs=[_rows(n, HID), _full((HID, HID)), _full((1, HID))],
        out_specs=_rows(n, HID),
        out_shape=jax.ShapeDtypeStruct((n, HID), jnp.float32),
    )(x, w, b)


def _tc_update(x, m, wg, bg, wfx_next, bf_next):
    n = x.shape[0]

    def body(x_ref, m_ref, wg_ref, bg_ref, wn_ref, bn_ref, o1_ref, o2_ref):
        x1 = x_ref[...] + jnp.maximum(
            jnp.dot(m_ref[...], wg_ref[...], preferred_element_type=jnp.float32)
            + bg_ref[...],
            0.0,
        )
        o1_ref[...] = x1
        o2_ref[...] = (
            jnp.dot(x1, wn_ref[...], preferred_element_type=jnp.float32) + bn_ref[...]
        )

    return pl.pallas_call(
        body,
        grid=(1,),
        in_specs=[
            _rows(n, HID),
            _rows(n, HID),
            _full((HID, HID)),
            _full((1, HID)),
            _full((HID, HID)),
            _full((1, HID)),
        ],
        out_specs=[_rows(n, HID), _rows(n, HID)],
        out_shape=[
            jax.ShapeDtypeStruct((n, HID), jnp.float32),
            jax.ShapeDtypeStruct((n, HID), jnp.float32),
        ],
    )(x, m, wg, bg, wfx_next, bf_next)


def _tc_update_last(x, m, wg, bg):
    n = x.shape[0]

    def body(x_ref, m_ref, wg_ref, bg_ref, o_ref):
        o_ref[...] = x_ref[...] + jnp.maximum(
            jnp.dot(m_ref[...], wg_ref[...], preferred_element_type=jnp.float32)
            + bg_ref[...],
            0.0,
        )

    return pl.pallas_call(
        body,
        grid=(1,),
        in_specs=[_rows(n, HID), _rows(n, HID), _full((HID, HID)), _full((1, HID))],
        out_specs=_rows(n, HID),
        out_shape=jax.ShapeDtypeStruct((n, HID), jnp.float32),
    )(x, m, wg, bg)


def _tc_addmm(g, r, wb):
    n = g.shape[0]
    bm = 512

    def body(g_ref, r_ref, wb_ref, o_ref):
        o_ref[...] = g_ref[...] + jnp.dot(
            r_ref[...], wb_ref[...], preferred_element_type=jnp.float32
        )

    return pl.pallas_call(
        body,
        grid=(n // bm,),
        in_specs=[_rows(bm, HID), _rows(bm, 4), _full((4, HID))],
        out_specs=_rows(bm, HID),
        out_shape=jax.ShapeDtypeStruct((n, HID), jnp.float32),
    )(g, r, wb)


def _tc_final(g, r, wa, wb, b, wc, bc):
    n = g.shape[0]
    bm = 1024

    def body(g_ref, r_ref, wa_ref, wb_ref, b_ref, wc_ref, bc_ref, o_ref):
        f = jnp.maximum(
            jnp.dot(g_ref[...], wa_ref[...], preferred_element_type=jnp.float32)
            + jnp.dot(r_ref[...], wb_ref[...], preferred_element_type=jnp.float32)
            + b_ref[...],
            0.0,
        )
        o_ref[...] = (
            jnp.dot(f, wc_ref[...], preferred_element_type=jnp.float32) + bc_ref[...]
        )

    return pl.pallas_call(
        body,
        grid=(n // bm,),
        in_specs=[
            _rows(bm, HID),
            _rows(bm, 4),
            _full((HID, HID)),
            _full((4, HID)),
            _full((1, HID)),
            _full((HID, N_CLASSES)),
            _full((1, N_CLASSES)),
        ],
        out_specs=_rows(bm, N_CLASSES),
        out_shape=jax.ShapeDtypeStruct((n, N_CLASSES), jnp.float32),
    )(g, r, wa, wb, b, wc, bc)


def kernel(remission, points, l1_cluster_centers, l2_cluster_centers,
           l1_edges, l2_edges, l1_labels, l2_labels,
           W1a, b1a, W1b, b1b, W1c, b1c, W3, b3,
           Wf4, bf4, Wg4, bg4, Wf41, bf41, Wg41, bg41,
           W5, b5, W7, b7, Wc, bc):
    f32 = jnp.float32
    xin = jnp.zeros((NPp, 4), f32).at[:N_POINTS, 0:1].set(remission)
    xin = xin.at[:N_POINTS, 1:4].set(points)
    c1t = jnp.zeros((NL1p, 4), f32).at[:N_L1, 1:4].set(l1_cluster_centers)
    c2t = jnp.zeros((NL2p, 4), f32).at[:N_L2, 1:4].set(l2_cluster_centers)
    c1_4 = jnp.zeros((NL1p, 4), f32).at[:N_L1, 1:4].set(l1_cluster_centers)
    lbl1 = jnp.full((NPp,), N_L1, jnp.int32).at[:N_POINTS].set(l1_labels)
    lbl2_g = jnp.zeros((NL1p,), jnp.int32).at[:N_L1].set(l2_labels)
    lbl2_m = jnp.full((NL1p,), -1, jnp.int32).at[:N_L1].set(l2_labels)
    src = l2_edges[0]
    dst = l2_edges[1]
    lbl1_2d = lbl1.reshape(NPp // CH, CH)
    lbl2g_2d = lbl2_g.reshape(NL1p // CH, CH)
    src_2d = src.reshape(E_L2 // CH, CH)

    def split67(w):
        return w[:HID], jnp.concatenate([jnp.zeros((1, HID), f32), w[HID:]], axis=0)

    W3a, W3b = split67(W3)
    W5a, W5b = split67(W5)
    W7a, W7b = split67(W7)
    Wf4x, Wf4e = split67(Wf4)
    Wf41x, Wf41e = split67(Wf41)
    b1a_, b1b_, b1c_, b3_ = (b.reshape(1, HID) for b in (b1a, b1b, b1c, b3))
    bf4_, bg4_, bf41_, bg41_ = (b.reshape(1, HID) for b in (bf4, bg4, bf41, bg41))
    b5_, b7_ = b5.reshape(1, HID), b7.reshape(1, HID)
    bc_ = bc.reshape(1, N_CLASSES)

    rel_in = _sc_rel_label(NPp, NL1p, xin.reshape(-1), c1t.reshape(-1), lbl1)
    rel_in = rel_in.reshape(NPp, 4)
    h = _tc_encode(rel_in, W1a, b1a_)
    aggp = _sc_scatter_add(h, lbl1_2d)
    f1 = _tc_mlp2(aggp, W1b, b1b_, W1c, b1c_)

    rel2 = _sc_rel_label(NL1p, NL2p, c1t.reshape(-1), c2t.reshape(-1), lbl2_g)
    rel2 = rel2.reshape(NL1p, 4)
    m = _tc_mix(f1, rel2, W3a, W3b, b3_)
    x = _sc_segmax(m, lbl2_m)

    rel_e = _sc_rel_edges(E_L2, NL2p, c2t.reshape(-1), src, dst)
    rel_e = rel_e.reshape(E_L2, 4)

    p4 = _tc_matmul_bias(x, Wf4x, bf4_)
    z4 = _tc_addmm(_sc_gather_rows(p4, src_2d), rel_e, Wf4e)
    m4 = _sc_segmax(z4, dst)
    x, p41 = _tc_update(x, m4, Wg4, bg4_, Wf41x, bf41_)
    z41 = _tc_addmm(_sc_gather_rows(p41, src_2d), rel_e, Wf41e)
    m41 = _sc_segmax(z41, dst)
    x = _tc_update_last(x, m41, Wg41, bg41_)

    g4 = _sc_gather_rows(x, lbl2g_2d)
    f5 = _tc_mix(g4, rel2, W5a, W5b, b5_)
    g5 = _sc_gather_rows(f5, lbl1_2d)
    out = _tc_final(g5, rel_in, W7a, W7b, b7_, Wc, bc_)
    return out[:N_POINTS]


# speedup vs baseline: 1.0079x; 1.0010x over previous
import functools

import jax
import jax.numpy as jnp
from jax import lax
from jax.experimental import pallas as pl
from jax.experimental.pallas import tpu as pltpu
from jax.experimental.pallas import tpu_sc as plsc

N_POINTS = 100000
N_L1 = 20000
N_L2 = 4000
E_L2 = 64000
HID = 64
N_CLASSES = 20

NPp = 102400
NL1p = 20480
NL2p = 4096
NW = 32
CH = 128

_mesh = plsc.VectorSubcoreMesh(core_axis_name="c", subcore_axis_name="s")
_sc_params = pltpu.CompilerParams(
    needs_layout_passes=False, use_tc_tiling_on_sc=False
)


def _wid():
    return lax.axis_index("s") * 2 + lax.axis_index("c")


def _ci(n, cap=8):
    nch = n // CH
    k = 1
    for cand in range(2, cap + 1):
        if nch % cand == 0:
            k = cand
    return k * CH


def _zero_vmem(ref, rows):
    z16 = jnp.zeros((16,), jnp.float32)

    @pl.loop(0, rows)
    def _(r):
        for q in range(HID // 16):
            ref[r, pl.ds(q * 16, 16)] = z16


def _sc_rel_label(n, t, a_flat, tbl_flat, idx):
    ci = _ci(n)
    nchunks = n // ci

    @functools.partial(
        pl.kernel,
        out_type=jax.ShapeDtypeStruct((n * 4,), jnp.float32),
        mesh=_mesh,
        compiler_params=_sc_params,
        scratch_types=[
            pltpu.VMEM((t * 4,), jnp.float32),
            pltpu.VMEM((ci,), jnp.int32),
            pltpu.VMEM((ci * 4,), jnp.float32),
            pltpu.VMEM((ci * 4,), jnp.float32),
        ],
    )
    def k(a_hbm, tbl_hbm, idx_hbm, out_hbm, tblv, idxv, xv, ov):
        wid = _wid()
        pltpu.sync_copy(tbl_hbm, tblv)
        iota = lax.iota(jnp.int32, 16)
        idiv = lax.shift_right_logical(iota, 2)
        imod = lax.bitwise_and(iota, 3)

        @pl.loop(wid, nchunks, step=NW)
        def _(c):
            base = c * ci
            pltpu.sync_copy(idx_hbm.at[pl.ds(base, ci)], idxv)
            pltpu.sync_copy(a_hbm.at[pl.ds(base * 4, ci * 4)], xv)
            for v in range(ci * 4 // 16):
                rowsel = plsc.load_gather(idxv, [jnp.int32(v * 4) + idiv])
                tidx = rowsel * 4 + imod
                tv = plsc.load_gather(tblv, [tidx])
                ov[pl.ds(v * 16, 16)] = xv[pl.ds(v * 16, 16)] - tv
            pltpu.sync_copy(ov, out_hbm.at[pl.ds(base * 4, ci * 4)])

    return k(a_flat, tbl_flat, idx)


def _sc_rel_edges(n, t, tbl_flat, src, dst):
    ci = _ci(n)
    nchunks = n // ci

    @functools.partial(
        pl.kernel,
        out_type=jax.ShapeDtypeStruct((n * 4,), jnp.float32),
        mesh=_mesh,
        compiler_params=_sc_params,
        scratch_types=[
            pltpu.VMEM((t * 4,), jnp.float32),
            pltpu.VMEM((ci,), jnp.int32),
            pltpu.VMEM((ci,), jnp.int32),
            pltpu.VMEM((ci * 4,), jnp.float32),
        ],
    )
    def k(tbl_hbm, src_hbm, dst_hbm, out_hbm, tblv, sv, dv, ov):
        wid = _wid()
        pltpu.sync_copy(tbl_hbm, tblv)
        iota = lax.iota(jnp.int32, 16)
        idiv = lax.shift_right_logical(iota, 2)
        imod = lax.bitwise_and(iota, 3)

        @pl.loop(wid, nchunks, step=NW)
        def _(c):
            base = c * ci
            pltpu.sync_copy(src_hbm.at[pl.ds(base, ci)], sv)
            pltpu.sync_copy(dst_hbm.at[pl.ds(base, ci)], dv)
            for v in range(ci * 4 // 16):
                lane = jnp.int32(v * 4) + idiv
                si = plsc.load_gather(sv, [lane]) * 4 + imod
                di = plsc.load_gather(dv, [lane]) * 4 + imod
                ov[pl.ds(v * 16, 16)] = (
                    plsc.load_gather(tblv, [si]) - plsc.load_gather(tblv, [di])
                )
            pltpu.sync_copy(ov, out_hbm.at[pl.ds(base * 4, ci * 4)])

    return k(tbl_flat, src, dst)


def _sc_scatter_add(vals, idx2):
    n = vals.shape[0]
    ci = _ci(n, cap=4)
    kk = ci // CH
    nchunks = n // ci
    rows_per_sub = NL1p // 16

    @functools.partial(
        pl.kernel,
        out_type=jax.ShapeDtypeStruct((2, NL1p, HID), jnp.float32),
        mesh=_mesh,
        compiler_params=_sc_params,
        scratch_types=[
            pltpu.VMEM_SHARED((NL1p, HID), jnp.float32),
            pltpu.VMEM((ci, HID), jnp.float32),
            pltpu.VMEM((kk, CH), jnp.int32),
        ],
    )
    def k(vals_hbm, idx_hbm, out_hbm, acc, hv, idxv):
        cid = lax.axis_index("c")
        sid = lax.axis_index("s")
        wid = sid * 2 + cid
        _zero_vmem(hv, CH)
        for i in range(rows_per_sub // CH):
            pltpu.sync_copy(
                hv.at[pl.ds(0, CH), :],
                acc.at[pl.ds(sid * rows_per_sub + i * CH, CH), :],
            )
        plsc.subcore_barrier()

        @pl.loop(wid, nchunks, step=NW)
        def _(c):
            pltpu.sync_copy(idx_hbm.at[pl.ds(c * kk, kk), :], idxv)
            pltpu.sync_copy(vals_hbm.at[pl.ds(c * ci, ci), :], hv)
            for i in range(kk):
                pltpu.sync_copy(
                    hv.at[pl.ds(i * CH, CH), :], acc.at[idxv.at[i]], add=True
                )

        plsc.subcore_barrier()
        pltpu.sync_copy(
            acc.at[pl.ds(sid * rows_per_sub, rows_per_sub), :],
            out_hbm.at[cid, pl.ds(sid * rows_per_sub, rows_per_sub), :],
        )

    return k(vals, idx2)


def _sc_gather_rows(tbl, idx2):
    n = idx2.shape[0] * CH
    ci = _ci(n)
    kk = ci // CH
    nchunks = n // ci

    @functools.partial(
        pl.kernel,
        out_type=jax.ShapeDtypeStruct((n, HID), jnp.float32),
        mesh=_mesh,
        compiler_params=_sc_params,
        scratch_types=[
            pltpu.VMEM((kk, CH), jnp.int32),
            pltpu.VMEM((ci, HID), jnp.float32),
            pltpu.SemaphoreType.DMA,
        ],
    )
    def k(tbl_hbm, idx_hbm, out_hbm, idxv, rows, sem):
        wid = _wid()

        @pl.loop(wid, nchunks, step=NW)
        def _(c):
            pltpu.sync_copy(idx_hbm.at[pl.ds(c * kk, kk), :], idxv)
            descs = [
                pltpu.async_copy(
                    tbl_hbm.at[idxv.at[i]], rows.at[pl.ds(i * CH, CH), :], sem
                )
                for i in range(kk)
            ]
            for d in descs:
                d.wait()
            pltpu.sync_copy(rows, out_hbm.at[pl.ds(c * ci, ci), :])

    return k(tbl, idx2)


def _sc_segmax(vals, idx):
    n = vals.shape[0]
    ci = _ci(n, cap=8)
    nchunks = n // ci
    CAP = 2048

    @functools.partial(
        pl.kernel,
        out_type=jax.ShapeDtypeStruct((NL2p, HID), jnp.float32),
        mesh=_mesh,
        compiler_params=_sc_params,
        scratch_types=[
            pltpu.VMEM((CH + 16, HID), jnp.float32),
            pltpu.VMEM((CH, HID), jnp.float32),
            pltpu.VMEM((ci,), jnp.int32),
            pltpu.VMEM(((CAP + CH) // CH, CH), jnp.int32),
            pltpu.VMEM((CAP + 16,), jnp.int32),
            pltpu.SemaphoreType.DMA,
        ],
    )
    def k(vals_hbm, idx_hbm, out_hbm, acc, rbuf, idxv, mrow, mdst, sem):
        wid = _wid()
        lo = wid * CH
        _zero_vmem(acc, CH + 16)
        zi = jnp.zeros((16,), jnp.int32)
        for r in range((CAP + CH) // CH):
            for g in range(CH // 16):
                mrow[r, pl.ds(g * 16, 16)] = zi
        iota = lax.iota(jnp.int32, 16)
        trash = jnp.full((16,), CH, jnp.int32)

        def flush(cnt):
            mdst[pl.ds(cnt, 16)] = trash
            ng = lax.div(cnt + 16, 16)

            @pl.loop(0, lax.div(ng + 7, 8))
            def _(s):
                pltpu.async_copy(vals_hbm.at[mrow.at[s]], rbuf, sem).wait()
                gs = jnp.minimum(8, ng - s * 8)

                @pl.loop(0, gs)
                def _(g):
                    dv = mdst[pl.ds(s * CH + g * 16, 16)]
                    for j in range(16):
                        d = dv[j]
                        for q in range(HID // 16):
                            sl = pl.ds(q * 16, 16)
                            acc[d, sl] = jnp.maximum(
                                acc[d, sl], rbuf[g * 16 + j, sl]
                            )

        @pl.loop(0, nchunks, init_carry=jnp.int32(0))
        def scan(c, cnt_in):
            base = c * ci
            pltpu.sync_copy(idx_hbm.at[pl.ds(base, ci)], idxv)
            cntv = jnp.full((16,), cnt_in, jnp.int32)
            for v in range(ci // 16):
                lbl = idxv[pl.ds(v * 16, 16)]
                rel = lbl - lo
                cmask = (rel >= 0) & (rel < CH)
                pos = cntv + plsc.cumsum(cmask.astype(jnp.int32)) - 1
                rowid = base + v * 16 + iota
                plsc.store_scatter(
                    mrow,
                    [lax.shift_right_logical(pos, 7), lax.bitwise_and(pos, CH - 1)],
                    rowid,
                    mask=cmask,
                )
                plsc.store_scatter(mdst, [pos], rel, mask=cmask)
                cntv = cntv + plsc.all_reduce_population_count(cmask)
            cnt = cntv[0]
            do_flush = cnt > CAP - ci

            @pl.when(do_flush)
            def _():
                flush(cnt)

            return jnp.where(do_flush, jnp.int32(0), cnt)

        flush(scan)
        pltpu.sync_copy(acc.at[pl.ds(0, CH), :], out_hbm.at[pl.ds(lo, CH), :])

    return k(vals, idx)


def _full(shape):
    return pl.BlockSpec(shape, lambda i: tuple(0 for _ in shape))


def _rows(bm, ncol):
    return pl.BlockSpec((bm, ncol), lambda i: (i, 0))


def _tc_encode(x, w, b):
    n = x.shape[0]
    bm = 1024

    def body(x_ref, w_ref, b_ref, o_ref):
        o_ref[...] = jnp.maximum(
            jnp.dot(x_ref[...], w_ref[...], preferred_element_type=jnp.float32)
            + b_ref[...],
            0.0,
        )

    return pl.pallas_call(
        body,
        grid=(n // bm,),
        in_specs=[_rows(bm, 4), _full((4, HID)), _full((1, HID))],
        out_specs=_rows(bm, HID),
        out_shape=jax.ShapeDtypeStruct((n, HID), jnp.float32),
    )(x, w, b)


def _tc_mlp2(aggp, w1, b1, w2, b2):
    n = aggp.shape[1]
    bm = 512

    def body(a_ref, w1_ref, b1_ref, w2_ref, b2_ref, o_ref):
        x = a_ref[0] + a_ref[1]
        h = jnp.maximum(
            jnp.dot(x, w1_ref[...], preferred_element_type=jnp.float32) + b1_ref[...],
            0.0,
        )
        o_ref[...] = jnp.maximum(
            jnp.dot(h, w2_ref[...], preferred_element_type=jnp.float32) + b2_ref[...],
            0.0,
        )

    return pl.pallas_call(
        body,
        grid=(n // bm,),
        in_specs=[
            pl.BlockSpec((2, bm, HID), lambda i: (0, i, 0)),
            _full((HID, HID)),
            _full((1, HID)),
            _full((HID, HID)),
            _full((1, HID)),
        ],
        out_specs=_rows(bm, HID),
        out_shape=jax.ShapeDtypeStruct((n, HID), jnp.float32),
    )(aggp, w1, b1, w2, b2)


def _tc_mix(g, r, wa, wb, b, relu=True, bm=512):
    n = g.shape[0]

    def body(g_ref, r_ref, wa_ref, wb_ref, b_ref, o_ref):
        x = (
            jnp.dot(g_ref[...], wa_ref[...], preferred_element_type=jnp.float32)
            + jnp.dot(r_ref[...], wb_ref[...], preferred_element_type=jnp.float32)
            + b_ref[...]
        )
        o_ref[...] = jnp.maximum(x, 0.0) if relu else x

    return pl.pallas_call(
        body,
        grid=(n // bm,),
        in_specs=[
            _rows(bm, HID),
            _rows(bm, 4),
            _full((HID, HID)),
            _full((4, HID)),
            _full((1, HID)),
        ],
        out_specs=_rows(bm, HID),
        out_shape=jax.ShapeDtypeStruct((n, HID), jnp.float32),
    )(g, r, wa, wb, b)


def _tc_matmul_bias(x, w, b):
    n = x.shape[0]

    def body(x_ref, w_ref, b_ref, o_ref):
        o_ref[...] = (
            jnp.dot(x_ref[...], w_ref[...], preferred_element_type=jnp.float32)
            + b_ref[...]
        )

    return pl.pallas_call(
        body,
        grid=(1,),
        in_specs=[_rows(n, HID), _full((HID, HID)), _full((1, HID))],
        out_specs=_rows(n, HID),
        out_shape=jax.ShapeDtypeStruct((n, HID), jnp.float32),
    )(x, w, b)


def _tc_update(x, m, wg, bg, wfx_next, bf_next):
    n = x.shape[0]

    def body(x_ref, m_ref, wg_ref, bg_ref, wn_ref, bn_ref, o1_ref, o2_ref):
        x1 = x_ref[...] + jnp.maximum(
            jnp.dot(m_ref[...], wg_ref[...], preferred_element_type=jnp.float32)
            + bg_ref[...],
            0.0,
        )
        o1_ref[...] = x1
        o2_ref[...] = (
            jnp.dot(x1, wn_ref[...], preferred_element_type=jnp.float32) + bn_ref[...]
        )

    return pl.pallas_call(
        body,
        grid=(1,),
        in_specs=[
            _rows(n, HID),
            _rows(n, HID),
            _full((HID, HID)),
            _full((1, HID)),
            _full((HID, HID)),
            _full((1, HID)),
        ],
        out_specs=[_rows(n, HID), _rows(n, HID)],
        out_shape=[
            jax.ShapeDtypeStruct((n, HID), jnp.float32),
            jax.ShapeDtypeStruct((n, HID), jnp.float32),
        ],
    )(x, m, wg, bg, wfx_next, bf_next)


def _tc_update_last(x, m, wg, bg):
    n = x.shape[0]

    def body(x_ref, m_ref, wg_ref, bg_ref, o_ref):
        o_ref[...] = x_ref[...] + jnp.maximum(
            jnp.dot(m_ref[...], wg_ref[...], preferred_element_type=jnp.float32)
            + bg_ref[...],
            0.0,
        )

    return pl.pallas_call(
        body,
        grid=(1,),
        in_specs=[_rows(n, HID), _rows(n, HID), _full((HID, HID)), _full((1, HID))],
        out_specs=_rows(n, HID),
        out_shape=jax.ShapeDtypeStruct((n, HID), jnp.float32),
    )(x, m, wg, bg)


def _tc_addmm(g, r, wb):
    n = g.shape[0]
    bm = 512

    def body(g_ref, r_ref, wb_ref, o_ref):
        o_ref[...] = g_ref[...] + jnp.dot(
            r_ref[...], wb_ref[...], preferred_element_type=jnp.float32
        )

    return pl.pallas_call(
        body,
        grid=(n // bm,),
        in_specs=[_rows(bm, HID), _rows(bm, 4), _full((4, HID))],
        out_specs=_rows(bm, HID),
        out_shape=jax.ShapeDtypeStruct((n, HID), jnp.float32),
    )(g, r, wb)


def _tc_final(g, r, wa, wb, b, wc, bc):
    n = g.shape[0]
    bm = 1024

    def body(g_ref, r_ref, wa_ref, wb_ref, b_ref, wc_ref, bc_ref, o_ref):
        f = jnp.maximum(
            jnp.dot(g_ref[...], wa_ref[...], preferred_element_type=jnp.float32)
            + jnp.dot(r_ref[...], wb_ref[...], preferred_element_type=jnp.float32)
            + b_ref[...],
            0.0,
        )
        o_ref[...] = (
            jnp.dot(f, wc_ref[...], preferred_element_type=jnp.float32) + bc_ref[...]
        )

    return pl.pallas_call(
        body,
        grid=(n // bm,),
        in_specs=[
            _rows(bm, HID),
            _rows(bm, 4),
            _full((HID, HID)),
            _full((4, HID)),
            _full((1, HID)),
            _full((HID, N_CLASSES)),
            _full((1, N_CLASSES)),
        ],
        out_specs=_rows(bm, N_CLASSES),
        out_shape=jax.ShapeDtypeStruct((n, N_CLASSES), jnp.float32),
    )(g, r, wa, wb, b, wc, bc)


def kernel(remission, points, l1_cluster_centers, l2_cluster_centers,
           l1_edges, l2_edges, l1_labels, l2_labels,
           W1a, b1a, W1b, b1b, W1c, b1c, W3, b3,
           Wf4, bf4, Wg4, bg4, Wf41, bf41, Wg41, bg41,
           W5, b5, W7, b7, Wc, bc):
    f32 = jnp.float32
    xin = jnp.zeros((NPp, 4), f32).at[:N_POINTS, 0:1].set(remission)
    xin = xin.at[:N_POINTS, 1:4].set(points)
    c1t = jnp.zeros((NL1p, 4), f32).at[:N_L1, 1:4].set(l1_cluster_centers)
    c2t = jnp.zeros((NL2p, 4), f32).at[:N_L2, 1:4].set(l2_cluster_centers)
    c1_4 = jnp.zeros((NL1p, 4), f32).at[:N_L1, 1:4].set(l1_cluster_centers)
    lbl1 = jnp.full((NPp,), N_L1, jnp.int32).at[:N_POINTS].set(l1_labels)
    lbl2_g = jnp.zeros((NL1p,), jnp.int32).at[:N_L1].set(l2_labels)
    lbl2_m = jnp.full((NL1p,), -1, jnp.int32).at[:N_L1].set(l2_labels)
    src = l2_edges[0]
    dst = l2_edges[1]
    lbl1_2d = lbl1.reshape(NPp // CH, CH)
    lbl2g_2d = lbl2_g.reshape(NL1p // CH, CH)
    src_2d = src.reshape(E_L2 // CH, CH)

    def split67(w):
        return w[:HID], jnp.concatenate([jnp.zeros((1, HID), f32), w[HID:]], axis=0)

    W3a, W3b = split67(W3)
    W5a, W5b = split67(W5)
    W7a, W7b = split67(W7)
    Wf4x, Wf4e = split67(Wf4)
    Wf41x, Wf41e = split67(Wf41)
    b1a_, b1b_, b1c_, b3_ = (b.reshape(1, HID) for b in (b1a, b1b, b1c, b3))
    bf4_, bg4_, bf41_, bg41_ = (b.reshape(1, HID) for b in (bf4, bg4, bf41, bg41))
    b5_, b7_ = b5.reshape(1, HID), b7.reshape(1, HID)
    bc_ = bc.reshape(1, N_CLASSES)

    rel_in = _sc_rel_label(NPp, NL1p, xin.reshape(-1), c1t.reshape(-1), lbl1)
    rel_in = rel_in.reshape(NPp, 4)
    h = _tc_encode(rel_in, W1a, b1a_)
    aggp = _sc_scatter_add(h, lbl1_2d)
    f1 = _tc_mlp2(aggp, W1b, b1b_, W1c, b1c_)

    rel2 = _sc_rel_label(NL1p, NL2p, c1t.reshape(-1), c2t.reshape(-1), lbl2_g)
    rel2 = rel2.reshape(NL1p, 4)
    m = _tc_mix(f1, rel2, W3a, W3b, b3_)
    x = _sc_segmax(m, lbl2_m)

    rel_e = _sc_rel_edges(E_L2, NL2p, c2t.reshape(-1), src, dst)
    rel_e = rel_e.reshape(E_L2, 4)

    p4 = _tc_matmul_bias(x, Wf4x, bf4_)
    z4 = _tc_addmm(_sc_gather_rows(p4, src_2d), rel_e, Wf4e)
    m4 = _sc_segmax(z4, dst)
    x, p41 = _tc_update(x, m4, Wg4, bg4_, Wf41x, bf41_)
    z41 = _tc_addmm(_sc_gather_rows(p41, src_2d), rel_e, Wf41e)
    m41 = _sc_segmax(z41, dst)
    x = _tc_update_last(x, m41, Wg41, bg41_)

    g4 = _sc_gather_rows(x, lbl2g_2d)
    f5 = _tc_mix(g4, rel2, W5a, W5b, b5_)
    g5 = _sc_gather_rows(f5, lbl1_2d)
    out = _tc_final(g5, rel_in, W7a, W7b, b7_, Wc, bc_)
    return out[:N_POINTS]


# speedup vs baseline: 1.0614x; 1.0531x over previous
import functools

import jax
import jax.numpy as jnp
from jax import lax
from jax.experimental import pallas as pl
from jax.experimental.pallas import tpu as pltpu
from jax.experimental.pallas import tpu_sc as plsc

N_POINTS = 100000
N_L1 = 20000
N_L2 = 4000
E_L2 = 64000
HID = 64
N_CLASSES = 20

NPp = 102400
NL1p = 20480
NL2p = 4096
NW = 32
CH = 128

_mesh = plsc.VectorSubcoreMesh(core_axis_name="c", subcore_axis_name="s")
_sc_params = pltpu.CompilerParams(
    needs_layout_passes=False, use_tc_tiling_on_sc=False
)


def _wid():
    return lax.axis_index("s") * 2 + lax.axis_index("c")


def _ci(n, cap=8):
    nch = n // CH
    k = 1
    for cand in range(2, cap + 1):
        if nch % cand == 0:
            k = cand
    return k * CH


def _zero_vmem(ref, rows):
    z16 = jnp.zeros((16,), jnp.float32)

    @pl.loop(0, rows)
    def _(r):
        for q in range(HID // 16):
            ref[r, pl.ds(q * 16, 16)] = z16


def _sc_rel_label(n, t, a_flat, tbl_flat, idx):
    ci = _ci(n)
    nchunks = n // ci

    @functools.partial(
        pl.kernel,
        out_type=jax.ShapeDtypeStruct((n * 4,), jnp.float32),
        mesh=_mesh,
        compiler_params=_sc_params,
        scratch_types=[
            pltpu.VMEM((t * 4,), jnp.float32),
            pltpu.VMEM((ci,), jnp.int32),
            pltpu.VMEM((ci * 4,), jnp.float32),
            pltpu.VMEM((ci * 4,), jnp.float32),
        ],
    )
    def k(a_hbm, tbl_hbm, idx_hbm, out_hbm, tblv, idxv, xv, ov):
        wid = _wid()
        pltpu.sync_copy(tbl_hbm, tblv)
        iota = lax.iota(jnp.int32, 16)
        idiv = lax.shift_right_logical(iota, 2)
        imod = lax.bitwise_and(iota, 3)

        @pl.loop(wid, nchunks, step=NW)
        def _(c):
            base = c * ci
            pltpu.sync_copy(idx_hbm.at[pl.ds(base, ci)], idxv)
            pltpu.sync_copy(a_hbm.at[pl.ds(base * 4, ci * 4)], xv)
            for v in range(ci * 4 // 16):
                rowsel = plsc.load_gather(idxv, [jnp.int32(v * 4) + idiv])
                tidx = rowsel * 4 + imod
                tv = plsc.load_gather(tblv, [tidx])
                ov[pl.ds(v * 16, 16)] = xv[pl.ds(v * 16, 16)] - tv
            pltpu.sync_copy(ov, out_hbm.at[pl.ds(base * 4, ci * 4)])

    return k(a_flat, tbl_flat, idx)


def _sc_rel_edges(n, t, tbl_flat, src, dst):
    ci = _ci(n)
    nchunks = n // ci

    @functools.partial(
        pl.kernel,
        out_type=jax.ShapeDtypeStruct((n * 4,), jnp.float32),
        mesh=_mesh,
        compiler_params=_sc_params,
        scratch_types=[
            pltpu.VMEM((t * 4,), jnp.float32),
            pltpu.VMEM((ci,), jnp.int32),
            pltpu.VMEM((ci,), jnp.int32),
            pltpu.VMEM((ci * 4,), jnp.float32),
        ],
    )
    def k(tbl_hbm, src_hbm, dst_hbm, out_hbm, tblv, sv, dv, ov):
        wid = _wid()
        pltpu.sync_copy(tbl_hbm, tblv)
        iota = lax.iota(jnp.int32, 16)
        idiv = lax.shift_right_logical(iota, 2)
        imod = lax.bitwise_and(iota, 3)

        @pl.loop(wid, nchunks, step=NW)
        def _(c):
            base = c * ci
            pltpu.sync_copy(src_hbm.at[pl.ds(base, ci)], sv)
            pltpu.sync_copy(dst_hbm.at[pl.ds(base, ci)], dv)
            for v in range(ci * 4 // 16):
                lane = jnp.int32(v * 4) + idiv
                si = plsc.load_gather(sv, [lane]) * 4 + imod
                di = plsc.load_gather(dv, [lane]) * 4 + imod
                ov[pl.ds(v * 16, 16)] = (
                    plsc.load_gather(tblv, [si]) - plsc.load_gather(tblv, [di])
                )
            pltpu.sync_copy(ov, out_hbm.at[pl.ds(base * 4, ci * 4)])

    return k(tbl_flat, src, dst)


def _sc_scatter_add(vals, idx2):
    n = vals.shape[0]
    ci = _ci(n, cap=4)
    kk = ci // CH
    nchunks = n // ci
    rows_per_sub = NL1p // 16

    @functools.partial(
        pl.kernel,
        out_type=jax.ShapeDtypeStruct((2, NL1p, HID), jnp.float32),
        mesh=_mesh,
        compiler_params=_sc_params,
        scratch_types=[
            pltpu.VMEM_SHARED((NL1p, HID), jnp.float32),
            pltpu.VMEM((ci, HID), jnp.float32),
            pltpu.VMEM((kk, CH), jnp.int32),
        ],
    )
    def k(vals_hbm, idx_hbm, out_hbm, acc, hv, idxv):
        cid = lax.axis_index("c")
        sid = lax.axis_index("s")
        wid = sid * 2 + cid
        _zero_vmem(hv, CH)
        for i in range(rows_per_sub // CH):
            pltpu.sync_copy(
                hv.at[pl.ds(0, CH), :],
                acc.at[pl.ds(sid * rows_per_sub + i * CH, CH), :],
            )
        plsc.subcore_barrier()

        @pl.loop(wid, nchunks, step=NW)
        def _(c):
            pltpu.sync_copy(idx_hbm.at[pl.ds(c * kk, kk), :], idxv)
            pltpu.sync_copy(vals_hbm.at[pl.ds(c * ci, ci), :], hv)
            for i in range(kk):
                pltpu.sync_copy(
                    hv.at[pl.ds(i * CH, CH), :], acc.at[idxv.at[i]], add=True
                )

        plsc.subcore_barrier()
        pltpu.sync_copy(
            acc.at[pl.ds(sid * rows_per_sub, rows_per_sub), :],
            out_hbm.at[cid, pl.ds(sid * rows_per_sub, rows_per_sub), :],
        )

    return k(vals, idx2)


def _sc_gather_rows(tbl, idx2):
    n = idx2.shape[0] * CH
    ci = _ci(n)
    kk = ci // CH
    nchunks = n // ci

    @functools.partial(
        pl.kernel,
        out_type=jax.ShapeDtypeStruct((n, HID), jnp.float32),
        mesh=_mesh,
        compiler_params=_sc_params,
        scratch_types=[
            pltpu.VMEM((kk, CH), jnp.int32),
            pltpu.VMEM((ci, HID), jnp.float32),
            pltpu.SemaphoreType.DMA,
        ],
    )
    def k(tbl_hbm, idx_hbm, out_hbm, idxv, rows, sem):
        wid = _wid()

        @pl.loop(wid, nchunks, step=NW)
        def _(c):
            pltpu.sync_copy(idx_hbm.at[pl.ds(c * kk, kk), :], idxv)
            descs = [
                pltpu.async_copy(
                    tbl_hbm.at[idxv.at[i]], rows.at[pl.ds(i * CH, CH), :], sem
                )
                for i in range(kk)
            ]
            for d in descs:
                d.wait()
            pltpu.sync_copy(rows, out_hbm.at[pl.ds(c * ci, ci), :])

    return k(tbl, idx2)


def _sc_segmax(vals, idx):
    n = vals.shape[0]
    ci = _ci(n, cap=8)
    nchunks = n // ci
    CAP = 2048

    @functools.partial(
        pl.kernel,
        out_type=jax.ShapeDtypeStruct((NL2p, HID), jnp.float32),
        mesh=_mesh,
        compiler_params=_sc_params,
        scratch_types=[
            pltpu.VMEM((CH + 16, HID), jnp.float32),
            pltpu.VMEM((CH, HID), jnp.float32),
            pltpu.VMEM((2, ci), jnp.int32),
            pltpu.VMEM(((CAP + CH) // CH, CH), jnp.int32),
            pltpu.VMEM((CAP + 16,), jnp.int32),
            pltpu.SemaphoreType.DMA,
            pltpu.SemaphoreType.DMA,
        ],
    )
    def k(vals_hbm, idx_hbm, out_hbm, acc, rbuf, idxv, mrow, mdst, sem, semi):
        wid = _wid()
        lo = wid * CH
        _zero_vmem(acc, CH + 16)
        zi = jnp.zeros((16,), jnp.int32)
        for r in range((CAP + CH) // CH):
            for g in range(CH // 16):
                mrow[r, pl.ds(g * 16, 16)] = zi
        iota = lax.iota(jnp.int32, 16)
        trash = jnp.full((16,), CH, jnp.int32)

        def flush(cnt):
            mdst[pl.ds(cnt, 16)] = trash
            ng = lax.div(cnt + 16, 16)

            @pl.loop(0, lax.div(ng + 7, 8))
            def _(s):
                pltpu.async_copy(vals_hbm.at[mrow.at[s]], rbuf, sem).wait()
                gs = jnp.minimum(8, ng - s * 8)

                @pl.loop(0, gs)
                def _(g):
                    dv = mdst[pl.ds(s * CH + g * 16, 16)]
                    for j in range(16):
                        d = dv[j]
                        for q in range(HID // 16):
                            sl = pl.ds(q * 16, 16)
                            acc[d, sl] = jnp.maximum(
                                acc[d, sl], rbuf[g * 16 + j, sl]
                            )

        def scan_windows(buf, c, cnt_in):
            base = c * ci
            cntv = jnp.full((16,), cnt_in, jnp.int32)
            for v in range(ci // 16):
                lbl = buf[pl.ds(v * 16, 16)]
                rel = lbl - lo
                cmask = (rel >= 0) & (rel < CH)
                pos = cntv + plsc.cumsum(cmask.astype(jnp.int32)) - 1
                rowid = base + v * 16 + iota
                plsc.store_scatter(
                    mrow,
                    [lax.shift_right_logical(pos, 7), lax.bitwise_and(pos, CH - 1)],
                    rowid,
                    mask=cmask,
                )
                plsc.store_scatter(mdst, [pos], rel, mask=cmask)
                cntv = cntv + plsc.all_reduce_population_count(cmask)
            cnt = cntv[0]
            do_flush = cnt > CAP - ci

            @pl.when(do_flush)
            def _():
                flush(cnt)

            return jnp.where(do_flush, jnp.int32(0), cnt)

        def idx_wait(c, buf):
            pltpu.make_async_copy(
                idx_hbm.at[pl.ds(c * ci, ci)], buf, semi
            ).wait()

        pltpu.async_copy(idx_hbm.at[pl.ds(0, ci)], idxv.at[0], semi)

        @pl.loop(0, nchunks, step=2, init_carry=jnp.int32(0))
        def scan(c, cnt):
            idx_wait(c, idxv.at[0])
            pltpu.async_copy(
                idx_hbm.at[pl.ds((c + 1) * ci, ci)], idxv.at[1], semi
            )
            cnt = scan_windows(idxv.at[0], c, cnt)
            idx_wait(c + 1, idxv.at[1])

            @pl.when(c + 2 < nchunks)
            def _():
                pltpu.async_copy(
                    idx_hbm.at[pl.ds((c + 2) * ci, ci)], idxv.at[0], semi
                )

            return scan_windows(idxv.at[1], c + 1, cnt)

        flush(scan)
        pltpu.sync_copy(acc.at[pl.ds(0, CH), :], out_hbm.at[pl.ds(lo, CH), :])

    return k(vals, idx)


def _full(shape):
    return pl.BlockSpec(shape, lambda i: tuple(0 for _ in shape))


def _rows(bm, ncol):
    return pl.BlockSpec((bm, ncol), lambda i: (i, 0))


def _tc_encode(x, w, b):
    n = x.shape[0]
    bm = 1024

    def body(x_ref, w_ref, b_ref, o_ref):
        o_ref[...] = jnp.maximum(
            jnp.dot(x_ref[...], w_ref[...], preferred_element_type=jnp.float32)
            + b_ref[...],
            0.0,
        )

    return pl.pallas_call(
        body,
        grid=(n // bm,),
        in_specs=[_rows(bm, 4), _full((4, HID)), _full((1, HID))],
        out_specs=_rows(bm, HID),
        out_shape=jax.ShapeDtypeStruct((n, HID), jnp.float32),
    )(x, w, b)


def _tc_mlp2(aggp, w1, b1, w2, b2):
    n = aggp.shape[1]
    bm = 512

    def body(a_ref, w1_ref, b1_ref, w2_ref, b2_ref, o_ref):
        x = a_ref[0] + a_ref[1]
        h = jnp.maximum(
            jnp.dot(x, w1_ref[...], preferred_element_type=jnp.float32) + b1_ref[...],
            0.0,
        )
        o_ref[...] = jnp.maximum(
            jnp.dot(h, w2_ref[...], preferred_element_type=jnp.float32) + b2_ref[...],
            0.0,
        )

    return pl.pallas_call(
        body,
        grid=(n // bm,),
        in_specs=[
            pl.BlockSpec((2, bm, HID), lambda i: (0, i, 0)),
            _full((HID, HID)),
            _full((1, HID)),
            _full((HID, HID)),
            _full((1, HID)),
        ],
        out_specs=_rows(bm, HID),
        out_shape=jax.ShapeDtypeStruct((n, HID), jnp.float32),
    )(aggp, w1, b1, w2, b2)


def _tc_mix(g, r, wa, wb, b, relu=True, bm=512):
    n = g.shape[0]

    def body(g_ref, r_ref, wa_ref, wb_ref, b_ref, o_ref):
        x = (
            jnp.dot(g_ref[...], wa_ref[...], preferred_element_type=jnp.float32)
            + jnp.dot(r_ref[...], wb_ref[...], preferred_element_type=jnp.float32)
            + b_ref[...]
        )
        o_ref[...] = jnp.maximum(x, 0.0) if relu else x

    return pl.pallas_call(
        body,
        grid=(n // bm,),
        in_specs=[
            _rows(bm, HID),
            _rows(bm, 4),
            _full((HID, HID)),
            _full((4, HID)),
            _full((1, HID)),
        ],
        out_specs=_rows(bm, HID),
        out_shape=jax.ShapeDtypeStruct((n, HID), jnp.float32),
    )(g, r, wa, wb, b)


def _tc_matmul_bias(x, w, b):
    n = x.shape[0]

    def body(x_ref, w_ref, b_ref, o_ref):
        o_ref[...] = (
            jnp.dot(x_ref[...], w_ref[...], preferred_element_type=jnp.float32)
            + b_ref[...]
        )

    return pl.pallas_call(
        body,
        grid=(1,),
        in_specs=[_rows(n, HID), _full((HID, HID)), _full((1, HID))],
        out_specs=_rows(n, HID),
        out_shape=jax.ShapeDtypeStruct((n, HID), jnp.float32),
    )(x, w, b)


def _tc_update(x, m, wg, bg, wfx_next, bf_next):
    n = x.shape[0]

    def body(x_ref, m_ref, wg_ref, bg_ref, wn_ref, bn_ref, o1_ref, o2_ref):
        x1 = x_ref[...] + jnp.maximum(
            jnp.dot(m_ref[...], wg_ref[...], preferred_element_type=jnp.float32)
            + bg_ref[...],
            0.0,
        )
        o1_ref[...] = x1
        o2_ref[...] = (
            jnp.dot(x1, wn_ref[...], preferred_element_type=jnp.float32) + bn_ref[...]
        )

    return pl.pallas_call(
        body,
        grid=(1,),
        in_specs=[
            _rows(n, HID),
            _rows(n, HID),
            _full((HID, HID)),
            _full((1, HID)),
            _full((HID, HID)),
            _full((1, HID)),
        ],
        out_specs=[_rows(n, HID), _rows(n, HID)],
        out_shape=[
            jax.ShapeDtypeStruct((n, HID), jnp.float32),
            jax.ShapeDtypeStruct((n, HID), jnp.float32),
        ],
    )(x, m, wg, bg, wfx_next, bf_next)


def _tc_update_last(x, m, wg, bg):
    n = x.shape[0]

    def body(x_ref, m_ref, wg_ref, bg_ref, o_ref):
        o_ref[...] = x_ref[...] + jnp.maximum(
            jnp.dot(m_ref[...], wg_ref[...], preferred_element_type=jnp.float32)
            + bg_ref[...],
            0.0,
        )

    return pl.pallas_call(
        body,
        grid=(1,),
        in_specs=[_rows(n, HID), _rows(n, HID), _full((HID, HID)), _full((1, HID))],
        out_specs=_rows(n, HID),
        out_shape=jax.ShapeDtypeStruct((n, HID), jnp.float32),
    )(x, m, wg, bg)


def _tc_addmm(g, r, wb):
    n = g.shape[0]
    bm = 512

    def body(g_ref, r_ref, wb_ref, o_ref):
        o_ref[...] = g_ref[...] + jnp.dot(
            r_ref[...], wb_ref[...], preferred_element_type=jnp.float32
        )

    return pl.pallas_call(
        body,
        grid=(n // bm,),
        in_specs=[_rows(bm, HID), _rows(bm, 4), _full((4, HID))],
        out_specs=_rows(bm, HID),
        out_shape=jax.ShapeDtypeStruct((n, HID), jnp.float32),
    )(g, r, wb)


def _tc_final(g, r, wa, wb, b, wc, bc):
    n = g.shape[0]
    bm = 1024

    def body(g_ref, r_ref, wa_ref, wb_ref, b_ref, wc_ref, bc_ref, o_ref):
        f = jnp.maximum(
            jnp.dot(g_ref[...], wa_ref[...], preferred_element_type=jnp.float32)
            + jnp.dot(r_ref[...], wb_ref[...], preferred_element_type=jnp.float32)
            + b_ref[...],
            0.0,
        )
        o_ref[...] = (
            jnp.dot(f, wc_ref[...], preferred_element_type=jnp.float32) + bc_ref[...]
        )

    return pl.pallas_call(
        body,
        grid=(n // bm,),
        in_specs=[
            _rows(bm, HID),
            _rows(bm, 4),
            _full((HID, HID)),
            _full((4, HID)),
            _full((1, HID)),
            _full((HID, N_CLASSES)),
            _full((1, N_CLASSES)),
        ],
        out_specs=_rows(bm, N_CLASSES),
        out_shape=jax.ShapeDtypeStruct((n, N_CLASSES), jnp.float32),
    )(g, r, wa, wb, b, wc, bc)


def kernel(remission, points, l1_cluster_centers, l2_cluster_centers,
           l1_edges, l2_edges, l1_labels, l2_labels,
           W1a, b1a, W1b, b1b, W1c, b1c, W3, b3,
           Wf4, bf4, Wg4, bg4, Wf41, bf41, Wg41, bg41,
           W5, b5, W7, b7, Wc, bc):
    f32 = jnp.float32
    xin = jnp.zeros((NPp, 4), f32).at[:N_POINTS, 0:1].set(remission)
    xin = xin.at[:N_POINTS, 1:4].set(points)
    c1t = jnp.zeros((NL1p, 4), f32).at[:N_L1, 1:4].set(l1_cluster_centers)
    c2t = jnp.zeros((NL2p, 4), f32).at[:N_L2, 1:4].set(l2_cluster_centers)
    c1_4 = jnp.zeros((NL1p, 4), f32).at[:N_L1, 1:4].set(l1_cluster_centers)
    lbl1 = jnp.full((NPp,), N_L1, jnp.int32).at[:N_POINTS].set(l1_labels)
    lbl2_g = jnp.zeros((NL1p,), jnp.int32).at[:N_L1].set(l2_labels)
    lbl2_m = jnp.full((NL1p,), -1, jnp.int32).at[:N_L1].set(l2_labels)
    src = l2_edges[0]
    dst = l2_edges[1]
    lbl1_2d = lbl1.reshape(NPp // CH, CH)
    lbl2g_2d = lbl2_g.reshape(NL1p // CH, CH)
    src_2d = src.reshape(E_L2 // CH, CH)

    def split67(w):
        return w[:HID], jnp.concatenate([jnp.zeros((1, HID), f32), w[HID:]], axis=0)

    W3a, W3b = split67(W3)
    W5a, W5b = split67(W5)
    W7a, W7b = split67(W7)
    Wf4x, Wf4e = split67(Wf4)
    Wf41x, Wf41e = split67(Wf41)
    b1a_, b1b_, b1c_, b3_ = (b.reshape(1, HID) for b in (b1a, b1b, b1c, b3))
    bf4_, bg4_, bf41_, bg41_ = (b.reshape(1, HID) for b in (bf4, bg4, bf41, bg41))
    b5_, b7_ = b5.reshape(1, HID), b7.reshape(1, HID)
    bc_ = bc.reshape(1, N_CLASSES)

    rel_in = _sc_rel_label(NPp, NL1p, xin.reshape(-1), c1t.reshape(-1), lbl1)
    rel_in = rel_in.reshape(NPp, 4)
    h = _tc_encode(rel_in, W1a, b1a_)
    aggp = _sc_scatter_add(h, lbl1_2d)
    f1 = _tc_mlp2(aggp, W1b, b1b_, W1c, b1c_)

    rel2 = _sc_rel_label(NL1p, NL2p, c1t.reshape(-1), c2t.reshape(-1), lbl2_g)
    rel2 = rel2.reshape(NL1p, 4)
    m = _tc_mix(f1, rel2, W3a, W3b, b3_)
    x = _sc_segmax(m, lbl2_m)

    rel_e = _sc_rel_edges(E_L2, NL2p, c2t.reshape(-1), src, dst)
    rel_e = rel_e.reshape(E_L2, 4)

    p4 = _tc_matmul_bias(x, Wf4x, bf4_)
    z4 = _tc_addmm(_sc_gather_rows(p4, src_2d), rel_e, Wf4e)
    m4 = _sc_segmax(z4, dst)
    x, p41 = _tc_update(x, m4, Wg4, bg4_, Wf41x, bf41_)
    z41 = _tc_addmm(_sc_gather_rows(p41, src_2d), rel_e, Wf41e)
    m41 = _sc_segmax(z41, dst)
    x = _tc_update_last(x, m41, Wg41, bg41_)

    g4 = _sc_gather_rows(x, lbl2g_2d)
    f5 = _tc_mix(g4, rel2, W5a, W5b, b5_)
    g5 = _sc_gather_rows(f5, lbl1_2d)
    out = _tc_final(g5, rel_in, W7a, W7b, b7_, Wc, bc_)
    return out[:N_POINTS]
